# baseline (speedup 1.0000x reference)
import functools

import jax
import jax.numpy as jnp
from jax import lax
from jax.experimental import pallas as pl
from jax.experimental.pallas import tpu as pltpu
from jax.experimental.pallas import tpu_sc as plsc

_NUM_CLASSES = 1000
_NW = 32

_B, _C = 16384, 26
_N = _B * _C
_W = _N * _NUM_CLASSES
_RPW = _N // _NW
_WPW = _RPW * _NUM_CLASSES

_ZROWS = 104
_ZCHUNK = _ZROWS * _NUM_CLASSES
_NZ = _WPW // _ZCHUNK
_ZDEPTH = 4

_ICHUNK = 1024
_NI = _RPW // _ICHUNK
_PCOLS = 128
_PROWS = _RPW // _PCOLS
_SDEPTH = 4


def _onehot_kernel(in_hbm, out_hbm, zbuf, idxbuf, posbuf, ones, zsem, isem,
                   ssem):
  cid = lax.axis_index("c")
  sid = lax.axis_index("s")
  wid = sid * 2 + cid
  wbase = wid * _WPW
  rbase = wid * _RPW

  zero16 = jnp.zeros((16,), jnp.int32)
  one16 = jnp.ones((16,), jnp.int32)

  def _zb(i, c):
    zbuf[pl.ds(i * 16, 16)] = zero16
    return c

  lax.fori_loop(0, _ZCHUNK // 16, _zb, 0)
  for v in range(_PCOLS // 16):
    ones[pl.ds(v * 16, 16)] = one16

  def _zstart(i):
    pltpu.make_async_copy(
        zbuf, out_hbm.at[pl.ds(wbase + i * _ZCHUNK, _ZCHUNK)], zsem).start()

  def _zwait():
    pltpu.make_async_copy(
        zbuf, out_hbm.at[pl.ds(wbase, _ZCHUNK)], zsem).wait()

  for i in range(_ZDEPTH):
    _zstart(i)

  def _za(i, c):
    @pl.when(i + _ZDEPTH < _NZ)
    def _():
      _zstart(i + _ZDEPTH)

    _zwait()
    return c

  lax.fori_loop(0, _NZ - _ZDEPTH, _za, 0)

  iota16 = lax.iota(jnp.int32, 16)

  def _pb(c, carry):
    cp = pltpu.make_async_copy(
        in_hbm.at[pl.ds(rbase + c * _ICHUNK, _ICHUNK)], idxbuf, isem)
    cp.start()
    cp.wait()
    for v in range(_ICHUNK // 16):
      vec = idxbuf[pl.ds(v * 16, 16)]
      row = rbase + c * _ICHUNK + v * 16
      pos = (row + iota16) * _NUM_CLASSES + vec
      prow = c * (_ICHUNK // _PCOLS) + v // (_PCOLS // 16)
      pcol = (v % (_PCOLS // 16)) * 16
      posbuf[prow, pl.ds(pcol, 16)] = pos
    return carry

  lax.fori_loop(0, _NI, _pb, 0)

  def _zd(i, c):
    _zwait()
    return c

  lax.fori_loop(0, _ZDEPTH, _zd, 0)

  def _sstart(j):
    pltpu.make_async_copy(ones, out_hbm.at[posbuf.at[j]], ssem).start()

  def _swait(j):
    pltpu.make_async_copy(ones, out_hbm.at[posbuf.at[j]], ssem).wait()

  for j in range(_SDEPTH):
    _sstart(j)

  def _sa(j, c):
    @pl.when(j + _SDEPTH < _PROWS)
    def _():
      _sstart(j + _SDEPTH)

    _swait(j)
    return c

  lax.fori_loop(0, _PROWS, _sa, 0)


@jax.jit
def _onehot(flat_idx):
  mesh = plsc.VectorSubcoreMesh(core_axis_name="c", subcore_axis_name="s")
  run = functools.partial(
      pl.kernel,
      mesh=mesh,
      out_type=jax.ShapeDtypeStruct((_W,), jnp.int32),
      scratch_types=[
          pltpu.VMEM((_ZCHUNK,), jnp.int32),
          pltpu.VMEM((_ICHUNK,), jnp.int32),
          pltpu.VMEM((_PROWS, _PCOLS), jnp.int32),
          pltpu.VMEM((_PCOLS,), jnp.int32),
          pltpu.SemaphoreType.DMA,
          pltpu.SemaphoreType.DMA,
          pltpu.SemaphoreType.DMA,
      ],
  )(_onehot_kernel)
  return run(flat_idx)


def kernel(input):
  flat = input.reshape(_N).astype(jnp.int32)
  return _onehot(flat).reshape(_B, _C, _NUM_CLASSES)


# speedup vs baseline: 5.5820x; 5.5820x over previous
import functools

import jax
import jax.numpy as jnp
from jax import lax
from jax.experimental import pallas as pl
from jax.experimental.pallas import tpu as pltpu
from jax.experimental.pallas import tpu_sc as plsc

_NUM_CLASSES = 1000
_NW = 32

_B, _C = 16384, 26
_N = _B * _C
_W = _N * _NUM_CLASSES
_CH = _C // 2
_BPW = _B // 16
_RPW = _BPW * _CH
_WPW = _W // _NW

_ZCHUNK = 102_400
_NZ = _WPW // _ZCHUNK
_ZDEPTH = 4

_PCOLS = 128
_PROWS = _RPW // _PCOLS
_SDEPTH = 4


def _onehot_kernel(in_hbm, out_hbm, zbuf, idxbuf, posbuf, ones, zsem, isem,
                   ssem):
  cid = lax.axis_index("c")
  sid = lax.axis_index("s")
  zbase = (cid * 16 + sid) * _WPW
  b0 = sid * _BPW
  c0 = cid * _CH

  zero16 = jnp.zeros((16,), jnp.int32)
  one16 = jnp.ones((16,), jnp.int32)

  def _zb(i, c):
    zbuf[pl.ds(i * 16, 16)] = zero16
    return c

  lax.fori_loop(0, _ZCHUNK // 16, _zb, 0)
  for v in range(_PCOLS // 16):
    ones[pl.ds(v * 16, 16)] = one16

  for lc in range(_CH):
    pltpu.make_async_copy(
        in_hbm.at[pl.ds((c0 + lc) * _B + b0, _BPW)],
        idxbuf.at[pl.ds(lc * _BPW, _BPW)], isem).start()

  def _zstart(i):
    pltpu.make_async_copy(
        zbuf, out_hbm.at[pl.ds(zbase + i * _ZCHUNK, _ZCHUNK)], zsem).start()

  def _zwait():
    pltpu.make_async_copy(
        zbuf, out_hbm.at[pl.ds(zbase, _ZCHUNK)], zsem).wait()

  for i in range(_ZDEPTH):
    _zstart(i)

  def _za(i, c):
    @pl.when(i + _ZDEPTH < _NZ)
    def _():
      _zstart(i + _ZDEPTH)

    _zwait()
    return c

  lax.fori_loop(0, _NZ - _ZDEPTH, _za, 0)

  for lc in range(_CH):
    pltpu.make_async_copy(
        in_hbm.at[pl.ds((c0 + lc) * _B + b0, _BPW)],
        idxbuf.at[pl.ds(lc * _BPW, _BPW)], isem).wait()
  iota16 = lax.iota(jnp.int32, 16)

  def _pb(q, carry):
    lc = q >> 6
    lb = (q & 63) * 16
    vec = idxbuf[pl.ds(q * 16, 16)]
    cc = c0 + lc
    bb = b0 + lb + iota16
    pos = (cc * (_B * _NUM_CLASSES) + ((vec >> 3) << 17) + ((bb >> 7) << 10)
           + ((vec & 7) << 7) + (bb & 127))
    posbuf[q >> 3, pl.ds((q & 7) * 16, 16)] = pos
    return carry

  lax.fori_loop(0, _RPW // 16, _pb, 0)

  def _zd(i, c):
    _zwait()
    return c

  lax.fori_loop(0, _ZDEPTH, _zd, 0)
  plsc.subcore_barrier()

  def _sstart(j):
    pltpu.make_async_copy(ones, out_hbm.at[posbuf.at[j]], ssem).start()

  def _swait(j):
    pltpu.make_async_copy(ones, out_hbm.at[posbuf.at[j]], ssem).wait()

  for j in range(_SDEPTH):
    _sstart(j)

  def _sa(j, c):
    @pl.when(j + _SDEPTH < _PROWS)
    def _():
      _sstart(j + _SDEPTH)

    _swait(j)
    return c

  lax.fori_loop(0, _PROWS, _sa, 0)


@jax.jit
def _onehot(flat_idx_t):
  mesh = plsc.VectorSubcoreMesh(core_axis_name="c", subcore_axis_name="s")
  run = functools.partial(
      pl.kernel,
      mesh=mesh,
      out_type=jax.ShapeDtypeStruct((_W,), jnp.int32),
      scratch_types=[
          pltpu.VMEM((_ZCHUNK,), jnp.int32),
          pltpu.VMEM((_RPW,), jnp.int32),
          pltpu.VMEM((_PROWS, _PCOLS), jnp.int32),
          pltpu.VMEM((_PCOLS,), jnp.int32),
          pltpu.SemaphoreType.DMA,
          pltpu.SemaphoreType.DMA,
          pltpu.SemaphoreType.DMA,
      ],
  )(_onehot_kernel)
  return run(flat_idx_t)


def kernel(input):
  flat_t = input.T.reshape(_N).astype(jnp.int32)
  out = _onehot(flat_t)
  out = out.reshape(_C, _NUM_CLASSES // 8, _B // 128, 8, 128)
  out = out.transpose(2, 4, 0, 1, 3)
  return out.reshape(_B, _C, _NUM_CLASSES)


# speedup vs baseline: 5.5854x; 1.0006x over previous
import functools

import jax
import jax.numpy as jnp
from jax import lax
from jax.experimental import pallas as pl
from jax.experimental.pallas import tpu as pltpu
from jax.experimental.pallas import tpu_sc as plsc

_NUM_CLASSES = 1000
_NW = 32

_B, _C = 16384, 26
_N = _B * _C
_W = _N * _NUM_CLASSES
_CH = _C // 2
_BPW = _B // 16
_RPW = _BPW * _CH
_WPW = _W // _NW

_ZCHUNK = 104_000
_NZ = _WPW // _ZCHUNK
_ZDEPTH = 8

_PCOLS = 128
_PROWS = _RPW // _PCOLS
_SDEPTH = 8


def _onehot_kernel(in_hbm, out_hbm, zbuf, idxbuf, posbuf, ones, zsem, isem,
                   ssem):
  cid = lax.axis_index("c")
  sid = lax.axis_index("s")
  zbase = (cid * 16 + sid) * _WPW
  b0 = sid * _BPW
  c0 = cid * _CH

  zero16 = jnp.zeros((16,), jnp.int32)
  one16 = jnp.ones((16,), jnp.int32)

  def _zb(i, c):
    zbuf[pl.ds(i * 16, 16)] = zero16
    return c

  lax.fori_loop(0, _ZCHUNK // 16, _zb, 0)
  for v in range(_PCOLS // 16):
    ones[pl.ds(v * 16, 16)] = one16

  for lc in range(_CH):
    pltpu.make_async_copy(
        in_hbm.at[pl.ds((c0 + lc) * _B + b0, _BPW)],
        idxbuf.at[pl.ds(lc * _BPW, _BPW)], isem).start()

  def _zstart(i):
    pltpu.make_async_copy(
        zbuf, out_hbm.at[pl.ds(zbase + i * _ZCHUNK, _ZCHUNK)], zsem).start()

  def _zwait():
    pltpu.make_async_copy(
        zbuf, out_hbm.at[pl.ds(zbase, _ZCHUNK)], zsem).wait()

  for i in range(_ZDEPTH):
    _zstart(i)

  def _za(i, c):
    @pl.when(i + _ZDEPTH < _NZ)
    def _():
      _zstart(i + _ZDEPTH)

    _zwait()
    return c

  lax.fori_loop(0, _NZ - _ZDEPTH, _za, 0)

  for lc in range(_CH):
    pltpu.make_async_copy(
        in_hbm.at[pl.ds((c0 + lc) * _B + b0, _BPW)],
        idxbuf.at[pl.ds(lc * _BPW, _BPW)], isem).wait()
  iota16 = lax.iota(jnp.int32, 16)

  def _pb(q, carry):
    lc = q >> 6
    lb = (q & 63) * 16
    vec = idxbuf[pl.ds(q * 16, 16)]
    cc = c0 + lc
    bb = b0 + lb + iota16
    pos = (cc * (_B * _NUM_CLASSES) + ((vec >> 3) << 17) + ((bb >> 7) << 10)
           + ((vec & 7) << 7) + (bb & 127))
    posbuf[q >> 3, pl.ds((q & 7) * 16, 16)] = pos
    return carry

  lax.fori_loop(0, _RPW // 16, _pb, 0)

  def _zd(i, c):
    _zwait()
    return c

  lax.fori_loop(0, _ZDEPTH, _zd, 0)
  plsc.subcore_barrier()

  def _sstart(j):
    pltpu.make_async_copy(ones, out_hbm.at[posbuf.at[j]], ssem).start()

  def _swait(j):
    pltpu.make_async_copy(ones, out_hbm.at[posbuf.at[j]], ssem).wait()

  for j in range(_SDEPTH):
    _sstart(j)

  def _sa(j, c):
    @pl.when(j + _SDEPTH < _PROWS)
    def _():
      _sstart(j + _SDEPTH)

    _swait(j)
    return c

  lax.fori_loop(0, _PROWS, _sa, 0)


@jax.jit
def _onehot(flat_idx_t):
  mesh = plsc.VectorSubcoreMesh(core_axis_name="c", subcore_axis_name="s")
  run = functools.partial(
      pl.kernel,
      mesh=mesh,
      out_type=jax.ShapeDtypeStruct((_W,), jnp.int32),
      scratch_types=[
          pltpu.VMEM((_ZCHUNK,), jnp.int32),
          pltpu.VMEM((_RPW,), jnp.int32),
          pltpu.VMEM((_PROWS, _PCOLS), jnp.int32),
          pltpu.VMEM((_PCOLS,), jnp.int32),
          pltpu.SemaphoreType.DMA,
          pltpu.SemaphoreType.DMA,
          pltpu.SemaphoreType.DMA,
      ],
  )(_onehot_kernel)
  return run(flat_idx_t)


def kernel(input):
  flat_t = input.T.reshape(_N).astype(jnp.int32)
  out = _onehot(flat_t)
  out = out.reshape(_C, _NUM_CLASSES // 8, _B // 128, 8, 128)
  out = out.transpose(2, 4, 0, 1, 3)
  return out.reshape(_B, _C, _NUM_CLASSES)
